# Initial kernel scaffold; baseline (speedup 1.0000x reference)
import jax
import jax.numpy as jnp
from jax.experimental import pallas as pl


def kernel(x, edge_index, mask):
    raise NotImplementedError("write your pallas kernel here")


# speedup vs baseline: 2.6779x; 2.6779x over previous
import functools

import jax
import jax.numpy as jnp
from jax import lax
from jax.experimental import pallas as pl
from jax.experimental.pallas import tpu as pltpu
from jax.experimental.pallas import tpu_sc as plsc

N = 10000
E = 160000
D = 256
ITERS = 40

NC = 2
NS = 16
NW = NC * NS
R = 320
N_PAD = NW * R
K = 128
E_CAP = E + NW * K
LANES = 16
SENT = N_PAD


def _build_edge_list(row, col, w, keep):
    key = jnp.where(keep, row, SENT).astype(jnp.int32)
    order = jnp.argsort(key)
    rs = key[order]
    cs = col[order]
    ws = jnp.where(keep, w, 0.0)[order]
    bounds = (jnp.arange(NW + 1, dtype=jnp.int32) * R).astype(rs.dtype)
    start = jnp.searchsorted(rs, bounds).astype(jnp.int32)
    cnt = start[1:] - start[:-1]
    pcnt = ((cnt + (K - 1)) // K) * K
    pstart = jnp.concatenate(
        [jnp.zeros((1,), jnp.int32), jnp.cumsum(pcnt).astype(jnp.int32)])
    widx = jnp.minimum(rs // R, NW)
    j = jnp.arange(E, dtype=jnp.int32)
    pos = pstart[widx] + (j - start[widx])
    pos = jnp.where(rs < SENT, pos, E_CAP)
    colp = jnp.zeros((E_CAP,), jnp.int32).at[pos].set(cs, mode="drop")
    rlp = jnp.zeros((E_CAP,), jnp.int32).at[pos].set(
        (rs - widx * R).astype(jnp.int32), mode="drop")
    wp = jnp.zeros((E_CAP,), jnp.float32).at[pos].set(ws, mode="drop")
    meta = jnp.zeros((NW, LANES), jnp.int32)
    meta = meta.at[:, 0].set(pstart[:NW]).at[:, 1].set(pcnt // K)
    return colp, rlp, wp, meta


def _spmm_body(init_hbm, src_hbm, col_hbm, rl_hbm, w_hbm, meta_hbm, out_hbm,
               meta_v, colc_v, rlc_v, wc_v, rows_v, acc_v, sem):
    wid = lax.axis_index("s") * NC + lax.axis_index("c")
    r0 = pl.multiple_of(wid * R, R)

    pltpu.sync_copy(meta_hbm, meta_v)
    mrow = meta_v[wid]
    base0 = mrow[0]
    nch = mrow[1]

    pltpu.sync_copy(init_hbm.at[pl.ds(r0, R)], acc_v)

    def chunk(ci, carry):
        base = pl.multiple_of(base0 + ci * K, K)
        pltpu.sync_copy(col_hbm.at[pl.ds(base, K)], colc_v)
        pltpu.sync_copy(rl_hbm.at[pl.ds(base, K)], rlc_v)
        pltpu.sync_copy(w_hbm.at[pl.ds(base, K)], wc_v)
        pltpu.async_copy(src_hbm.at[colc_v], rows_v, sem).wait()

        def group(g, c2):
            o = pl.multiple_of(g * LANES, LANES)
            rlv = rlc_v[pl.ds(o, LANES)]
            wv = wc_v[pl.ds(o, LANES)]
            for j in range(LANES):
                rl = rlv[j]
                wj = wv[j]
                for dv in range(D // LANES):
                    sl = pl.ds(dv * LANES, LANES)
                    plsc.addupdate(acc_v.at[rl, sl], rows_v[o + j, sl] * wj)
            return c2

        lax.fori_loop(0, K // LANES, group, carry, unroll=False)
        return carry

    lax.fori_loop(0, nch, chunk, 0, unroll=False)
    pltpu.sync_copy(acc_v, out_hbm.at[pl.ds(r0, R)])


def _make_spmm():
    mesh = plsc.VectorSubcoreMesh(
        core_axis_name="c", subcore_axis_name="s",
        num_cores=NC, num_subcores=NS)
    return functools.partial(
        pl.kernel,
        out_type=jax.ShapeDtypeStruct((N_PAD, D), jnp.float32),
        mesh=mesh,
        scratch_types=[
            pltpu.VMEM((NW, LANES), jnp.int32),
            pltpu.VMEM((K,), jnp.int32),
            pltpu.VMEM((K,), jnp.int32),
            pltpu.VMEM((K,), jnp.float32),
            pltpu.VMEM((K, D), jnp.float32),
            pltpu.VMEM((R, D), jnp.float32),
            pltpu.SemaphoreType.DMA,
        ],
    )(_spmm_body)


def kernel(x, edge_index, mask):
    row = edge_index[0].astype(jnp.int32)
    col = edge_index[1].astype(jnp.int32)

    deg = jnp.zeros((N,), jnp.float32).at[col].add(1.0)
    dinv = jnp.where(deg > 0, 1.0 / jnp.sqrt(jnp.maximum(deg, 1e-12)), 0.0)
    w = dinv[row] * dinv[col]
    mr = mask[row]
    mc = mask[col]

    uu = _build_edge_list(row, col, w, (~mr) & (~mc))
    um = _build_edge_list(row, col, w, (~mr) & mc)

    x_pad = jnp.zeros((N_PAD, D), jnp.float32).at[:N].set(x)
    zeros_pad = jnp.zeros((N_PAD, D), jnp.float32)

    spmm = _make_spmm()

    b = spmm(zeros_pad, x_pad, *um)

    def step(_, h):
        return spmm(b, h, *uu)

    h = lax.fori_loop(0, ITERS - 1, step, b)

    return jnp.where(mask[:, None], x, h[:N])


# speedup vs baseline: 3.1541x; 1.1778x over previous
import functools

import jax
import jax.numpy as jnp
from jax import lax
from jax.experimental import pallas as pl
from jax.experimental.pallas import tpu as pltpu
from jax.experimental.pallas import tpu_sc as plsc

N = 10000
E = 160000
D = 256
ITERS = 40

NC = 2
NS = 16
NW = NC * NS
R = 320
N_PAD = NW * R
K = 128
E_CAP = E + NW * K
LANES = 16
SENT = N_PAD


def _build_edge_list(row, col, w, keep):
    key = jnp.where(keep, row, SENT).astype(jnp.int32)
    order = jnp.argsort(key)
    rs = key[order]
    cs = col[order]
    ws = jnp.where(keep, w, 0.0)[order]
    bounds = (jnp.arange(NW + 1, dtype=jnp.int32) * R).astype(rs.dtype)
    start = jnp.searchsorted(rs, bounds).astype(jnp.int32)
    cnt = start[1:] - start[:-1]
    pcnt = ((cnt + (K - 1)) // K) * K
    pstart = jnp.concatenate(
        [jnp.zeros((1,), jnp.int32), jnp.cumsum(pcnt).astype(jnp.int32)])
    widx = jnp.minimum(rs // R, NW)
    j = jnp.arange(E, dtype=jnp.int32)
    pos = pstart[widx] + (j - start[widx])
    pos = jnp.where(rs < SENT, pos, E_CAP)
    colp = jnp.zeros((E_CAP,), jnp.int32).at[pos].set(cs, mode="drop")
    rlp = jnp.zeros((E_CAP,), jnp.int32).at[pos].set(
        (rs - widx * R).astype(jnp.int32), mode="drop")
    wp = jnp.zeros((E_CAP,), jnp.float32).at[pos].set(ws, mode="drop")
    meta = jnp.zeros((NW, LANES), jnp.int32)
    meta = meta.at[:, 0].set(pstart[:NW]).at[:, 1].set(pcnt // K)
    return colp, rlp, wp, meta


def _spmm_body(init_hbm, src_hbm, col_hbm, rl_hbm, w_hbm, meta_hbm, out_hbm,
               meta_v, colc_v, rlc_v, wc_v, rows_v, acc_v, sem):
    wid = lax.axis_index("s") * NC + lax.axis_index("c")
    r0 = pl.multiple_of(wid * R, R)

    pltpu.sync_copy(meta_hbm, meta_v)
    mrow = meta_v[wid]
    base0 = mrow[0]
    nch = mrow[1]

    pltpu.sync_copy(init_hbm.at[pl.ds(r0, R)], acc_v)

    def chunk(ci, carry):
        base = pl.multiple_of(base0 + ci * K, K)
        pltpu.sync_copy(col_hbm.at[pl.ds(base, K)], colc_v)
        pltpu.sync_copy(rl_hbm.at[pl.ds(base, K)], rlc_v)
        pltpu.sync_copy(w_hbm.at[pl.ds(base, K)], wc_v)
        pltpu.async_copy(src_hbm.at[colc_v], rows_v, sem).wait()

        @plsc.parallel_loop(0, K // LANES, unroll=2)
        def group(g):
            o = pl.multiple_of(g * LANES, LANES)
            rlv = rlc_v[pl.ds(o, LANES)]
            wv = wc_v[pl.ds(o, LANES)]
            for j in range(LANES):
                rl = rlv[j]
                wj = wv[j]
                vals = [rows_v[o + j, pl.ds(dv * LANES, LANES)] * wj
                        for dv in range(D // LANES)]
                for dv in range(D // LANES):
                    sl = pl.ds(dv * LANES, LANES)
                    plsc.addupdate(acc_v.at[rl, sl], vals[dv])

        return carry

    lax.fori_loop(0, nch, chunk, 0, unroll=False)
    pltpu.sync_copy(acc_v, out_hbm.at[pl.ds(r0, R)])


def _make_spmm():
    mesh = plsc.VectorSubcoreMesh(
        core_axis_name="c", subcore_axis_name="s",
        num_cores=NC, num_subcores=NS)
    return functools.partial(
        pl.kernel,
        out_type=jax.ShapeDtypeStruct((N_PAD, D), jnp.float32),
        mesh=mesh,
        scratch_types=[
            pltpu.VMEM((NW, LANES), jnp.int32),
            pltpu.VMEM((K,), jnp.int32),
            pltpu.VMEM((K,), jnp.int32),
            pltpu.VMEM((K,), jnp.float32),
            pltpu.VMEM((K, D), jnp.float32),
            pltpu.VMEM((R, D), jnp.float32),
            pltpu.SemaphoreType.DMA,
        ],
    )(_spmm_body)


def kernel(x, edge_index, mask):
    row = edge_index[0].astype(jnp.int32)
    col = edge_index[1].astype(jnp.int32)

    deg = jnp.zeros((N,), jnp.float32).at[col].add(1.0)
    dinv = jnp.where(deg > 0, 1.0 / jnp.sqrt(jnp.maximum(deg, 1e-12)), 0.0)
    w = dinv[row] * dinv[col]
    mr = mask[row]
    mc = mask[col]

    uu = _build_edge_list(row, col, w, (~mr) & (~mc))
    um = _build_edge_list(row, col, w, (~mr) & mc)

    x_pad = jnp.zeros((N_PAD, D), jnp.float32).at[:N].set(x)
    zeros_pad = jnp.zeros((N_PAD, D), jnp.float32)

    spmm = _make_spmm()

    b = spmm(zeros_pad, x_pad, *um)

    def step(_, h):
        return spmm(b, h, *uu)

    h = lax.fori_loop(0, ITERS - 1, step, b)

    return jnp.where(mask[:, None], x, h[:N])


# speedup vs baseline: 3.1580x; 1.0012x over previous
import functools

import jax
import jax.numpy as jnp
from jax import lax
from jax.experimental import pallas as pl
from jax.experimental.pallas import tpu as pltpu
from jax.experimental.pallas import tpu_sc as plsc

N = 10000
E = 160000
D = 256
ITERS = 40

NC = 2
NS = 16
NW = NC * NS
R = 320
N_PAD = NW * R
K = 128
E_CAP = E + NW * K
LANES = 16
SENT = N_PAD
W_SCALE = float(2 ** 30)


def _build_edge_list(row, col, w, keep):
    key = jnp.where(keep, row, SENT).astype(jnp.int32)
    order = jnp.argsort(key)
    rs = key[order]
    cs = col[order]
    ws = jnp.where(keep, w, 0.0)[order]
    bounds = (jnp.arange(NW + 1, dtype=jnp.int32) * R).astype(rs.dtype)
    start = jnp.searchsorted(rs, bounds).astype(jnp.int32)
    cnt = start[1:] - start[:-1]
    pcnt = ((cnt + (K - 1)) // K) * K
    pstart = jnp.concatenate(
        [jnp.zeros((1,), jnp.int32), jnp.cumsum(pcnt).astype(jnp.int32)])
    widx = jnp.minimum(rs // R, NW)
    j = jnp.arange(E, dtype=jnp.int32)
    pos = pstart[widx] + (j - start[widx])
    pos = jnp.where(rs < SENT, pos, E_CAP)
    colp = jnp.zeros((E_CAP,), jnp.int32).at[pos].set(cs, mode="drop")
    rlp = jnp.zeros((E_CAP,), jnp.int32).at[pos].set(
        (rs - widx * R).astype(jnp.int32), mode="drop")
    wb = jnp.round(
        jnp.zeros((E_CAP,), jnp.float32).at[pos].set(ws, mode="drop")
        * W_SCALE).astype(jnp.int32)
    edata = jnp.stack([colp.reshape(-1, K), rlp.reshape(-1, K),
                       wb.reshape(-1, K)], axis=1).reshape(-1)
    edata = jnp.concatenate([edata, jnp.zeros((2 * 3 * K,), jnp.int32)])
    meta = jnp.zeros((NW, LANES), jnp.int32)
    meta = meta.at[:, 0].set(pstart[:NW]).at[:, 1].set(pcnt // K)
    return edata, meta


HK = K // 2


def _spmm_body(init_hbm, src_hbm, edata_hbm, meta_hbm, out_hbm,
               meta_v, ec_v, rows_a, rows_b, acc_v, sem_a, sem_b, sem_e):
    wid = lax.axis_index("s") * NC + lax.axis_index("c")
    r0 = pl.multiple_of(wid * R, R)

    pltpu.sync_copy(meta_hbm, meta_v)
    mrow = meta_v[wid]
    base0 = mrow[0]
    nch = mrow[1]

    pltpu.sync_copy(init_hbm.at[pl.ds(r0, R)], acc_v)

    eb0 = pl.multiple_of(base0 * 3, 3 * K)

    def ec_chunk(ci, p):
        off = pl.multiple_of(eb0 + ci * (3 * K), 3 * K)
        return edata_hbm.at[pl.ds(off, 3 * K)], ec_v.at[p]

    def gather(p, ho, rows_ref, sem):
        idx = ec_v.at[p, pl.ds(ho, HK)]
        return pltpu.async_copy(src_hbm.at[idx], rows_ref, sem)

    def gather_wait(p, ho, rows_ref, sem):
        idx = ec_v.at[p, pl.ds(ho, HK)]
        pltpu.make_async_copy(src_hbm.at[idx], rows_ref, sem).wait()

    def process(rows_ref, p, ho):
        @plsc.parallel_loop(0, HK // LANES, unroll=2)
        def group(g):
            o = pl.multiple_of(g * LANES, LANES)
            rlv = ec_v[p, pl.ds(K + ho + o, LANES)]
            wv = (ec_v[p, pl.ds(2 * K + ho + o, LANES)]
                  .astype(jnp.float32) * (1.0 / W_SCALE))
            for j in range(LANES):
                rl = rlv[j]
                wj = wv[j]
                vals = [rows_ref[o + j, pl.ds(dv * LANES, LANES)] * wj
                        for dv in range(D // LANES)]
                for dv in range(D // LANES):
                    sl = pl.ds(dv * LANES, LANES)
                    plsc.addupdate(acc_v.at[rl, sl], vals[dv])

    pltpu.sync_copy(*ec_chunk(0, 0))
    gather(0, 0, rows_a, sem_a)
    gather(0, HK, rows_b, sem_b)
    pltpu.async_copy(*ec_chunk(1, 1), sem_e)

    def chunk(ci, carry):
        p = lax.rem(ci, 2)
        pn = 1 - p
        src_e, dst_e = ec_chunk(ci + 1, pn)
        pltpu.make_async_copy(src_e, dst_e, sem_e).wait()
        gather_wait(p, 0, rows_a, sem_a)
        process(rows_a, p, 0)
        gather(pn, 0, rows_a, sem_a)
        gather_wait(p, HK, rows_b, sem_b)
        process(rows_b, p, HK)
        gather(pn, HK, rows_b, sem_b)
        pltpu.async_copy(*ec_chunk(ci + 2, p), sem_e)
        return carry

    lax.fori_loop(0, nch, chunk, 0, unroll=False)

    gather_wait(0, 0, rows_a, sem_a)
    gather_wait(0, HK, rows_b, sem_b)
    pltpu.make_async_copy(*ec_chunk(0, 0), sem_e).wait()

    pltpu.sync_copy(acc_v, out_hbm.at[pl.ds(r0, R)])


def _make_spmm():
    mesh = plsc.VectorSubcoreMesh(
        core_axis_name="c", subcore_axis_name="s",
        num_cores=NC, num_subcores=NS)
    return functools.partial(
        pl.kernel,
        out_type=jax.ShapeDtypeStruct((N_PAD, D), jnp.float32),
        mesh=mesh,
        scratch_types=[
            pltpu.VMEM((NW, LANES), jnp.int32),
            pltpu.VMEM((2, 3 * K), jnp.int32),
            pltpu.VMEM((HK, D), jnp.float32),
            pltpu.VMEM((HK, D), jnp.float32),
            pltpu.VMEM((R, D), jnp.float32),
            pltpu.SemaphoreType.DMA,
            pltpu.SemaphoreType.DMA,
            pltpu.SemaphoreType.DMA,
        ],
    )(_spmm_body)


def kernel(x, edge_index, mask):
    row = edge_index[0].astype(jnp.int32)
    col = edge_index[1].astype(jnp.int32)

    deg = jnp.zeros((N,), jnp.float32).at[col].add(1.0)
    dinv = jnp.where(deg > 0, 1.0 / jnp.sqrt(jnp.maximum(deg, 1e-12)), 0.0)
    w = dinv[row] * dinv[col]
    mr = mask[row]
    mc = mask[col]

    uu = _build_edge_list(row, col, w, (~mr) & (~mc))
    um = _build_edge_list(row, col, w, (~mr) & mc)

    x_pad = jnp.zeros((N_PAD, D), jnp.float32).at[:N].set(x)
    zeros_pad = jnp.zeros((N_PAD, D), jnp.float32)

    spmm = _make_spmm()

    b = spmm(zeros_pad, x_pad, *um)

    def step(_, h):
        return spmm(b, h, *uu)

    h = lax.fori_loop(0, ITERS - 1, step, b)

    return jnp.where(mask[:, None], x, h[:N])


# speedup vs baseline: 5.5972x; 1.7724x over previous
import functools

import jax
import jax.numpy as jnp
from jax import lax
from jax.experimental import pallas as pl
from jax.experimental.pallas import tpu as pltpu
from jax.experimental.pallas import tpu_sc as plsc

N = 10000
E = 160000
D = 256
ITERS = 40

NC = 2
NS = 16
NW = NC * NS
R = 320
N_PAD = NW * R
K = 128
E_CAP = E + NW * K
LANES = 16
SENT = N_PAD
W_SCALE = float(2 ** 30)


def _build_lists(row, col, w, mask):
    mr = mask[row]
    mc = mask[col]
    uu = (~mr) & (~mc)
    um = (~mr) & mc
    key = jnp.where(uu, row,
                    jnp.where(um, N_PAD + row, 2 * N_PAD)).astype(jnp.int32)
    w30 = jnp.round(w * W_SCALE).astype(jnp.int32)
    rs, cs, ws = jax.lax.sort((key, col, w30), num_keys=1)
    pad = jnp.full((2 * K,), 2 * N_PAD, jnp.int32)
    rs = jnp.concatenate([rs, pad])
    cs = jnp.concatenate([cs, jnp.zeros((2 * K,), jnp.int32)])
    ws = jnp.concatenate([ws, jnp.zeros((2 * K,), jnp.int32)])

    def meta_for(base):
        bounds = base + jnp.arange(NW + 1, dtype=jnp.int32) * R
        st = jnp.searchsorted(rs, bounds).astype(jnp.int32)
        s, e = st[:NW], st[1:]
        a0 = (s // K) * K
        nch = jnp.where(e > s, (e - a0 + (K - 1)) // K, 0)
        meta = jnp.zeros((NW, LANES), jnp.int32)
        meta = (meta.at[:, 0].set(a0).at[:, 1].set(nch)
                .at[:, 2].set(s).at[:, 3].set(e)
                .at[:, 4].set(bounds[:NW]))
        return meta

    return rs, cs, ws, meta_for(0), meta_for(N_PAD)


HK = K // 2


def _spmm_body(init_hbm, src_hbm, cs_hbm, rs_hbm, ws_hbm, meta_hbm, out_hbm,
               meta_v, ec_v, rows_a, rows_b, acc_v, sem_a, sem_b, sem_e):
    wid = lax.axis_index("s") * NC + lax.axis_index("c")
    r0 = pl.multiple_of(wid * R, R)

    pltpu.sync_copy(meta_hbm, meta_v)
    mrow = meta_v[wid]
    base0 = mrow[0]
    nch = mrow[1]
    seg_s = mrow[2]
    seg_e = mrow[3]
    keyb = mrow[4]

    pltpu.sync_copy(init_hbm.at[pl.ds(r0, R)], acc_v)

    lane = jnp.arange(LANES, dtype=jnp.int32)

    def ec_copies(ci, p):
        off = pl.multiple_of(base0 + ci * K, K)
        return ((cs_hbm.at[pl.ds(off, K)], ec_v.at[p, pl.ds(0, K)]),
                (rs_hbm.at[pl.ds(off, K)], ec_v.at[p, pl.ds(K, K)]),
                (ws_hbm.at[pl.ds(off, K)], ec_v.at[p, pl.ds(2 * K, K)]))

    def ec_start(ci, p):
        for src, dst in ec_copies(ci, p):
            pltpu.async_copy(src, dst, sem_e)

    def ec_wait(ci, p):
        for src, dst in ec_copies(ci, p):
            pltpu.make_async_copy(src, dst, sem_e).wait()

    def gather(p, ho, rows_ref, sem):
        idx = ec_v.at[p, pl.ds(ho, HK)]
        return pltpu.async_copy(src_hbm.at[idx], rows_ref, sem)

    def gather_wait(p, ho, rows_ref, sem):
        idx = ec_v.at[p, pl.ds(ho, HK)]
        pltpu.make_async_copy(src_hbm.at[idx], rows_ref, sem).wait()

    def process(rows_ref, p, ho, base):
        @plsc.parallel_loop(0, HK // LANES, unroll=2)
        def group(g):
            o = pl.multiple_of(g * LANES, LANES)
            jv = (base + ho + o) + lane
            valid = (jv >= seg_s) & (jv < seg_e)
            rsv = ec_v[p, pl.ds(K + ho + o, LANES)]
            rlv = jnp.minimum(jnp.maximum(rsv - keyb, 0), R - 1)
            wv = jnp.where(
                valid,
                ec_v[p, pl.ds(2 * K + ho + o, LANES)].astype(jnp.float32)
                * (1.0 / W_SCALE),
                0.0)
            for j in range(LANES):
                rl = rlv[j]
                wj = wv[j]
                vals = [rows_ref[o + j, pl.ds(dv * LANES, LANES)] * wj
                        for dv in range(D // LANES)]
                for dv in range(D // LANES):
                    sl = pl.ds(dv * LANES, LANES)
                    plsc.addupdate(acc_v.at[rl, sl], vals[dv])

    for src, dst in ec_copies(0, 0):
        pltpu.sync_copy(src, dst)
    gather(0, 0, rows_a, sem_a)
    gather(0, HK, rows_b, sem_b)
    ec_start(1, 1)

    def chunk(ci, carry):
        p = lax.rem(ci, 2)
        pn = 1 - p
        base = pl.multiple_of(base0 + ci * K, K)
        ec_wait(ci + 1, pn)
        gather_wait(p, 0, rows_a, sem_a)
        process(rows_a, p, 0, base)
        gather(pn, 0, rows_a, sem_a)
        gather_wait(p, HK, rows_b, sem_b)
        process(rows_b, p, HK, base)
        gather(pn, HK, rows_b, sem_b)
        ec_start(ci + 2, p)
        return carry

    lax.fori_loop(0, nch, chunk, 0, unroll=False)

    gather_wait(0, 0, rows_a, sem_a)
    gather_wait(0, HK, rows_b, sem_b)
    ec_wait(0, 0)

    pltpu.sync_copy(acc_v, out_hbm.at[pl.ds(r0, R)])


def _make_spmm():
    mesh = plsc.VectorSubcoreMesh(
        core_axis_name="c", subcore_axis_name="s",
        num_cores=NC, num_subcores=NS)
    return functools.partial(
        pl.kernel,
        out_type=jax.ShapeDtypeStruct((N_PAD, D), jnp.float32),
        mesh=mesh,
        scratch_types=[
            pltpu.VMEM((NW, LANES), jnp.int32),
            pltpu.VMEM((2, 3 * K), jnp.int32),
            pltpu.VMEM((HK, D), jnp.float32),
            pltpu.VMEM((HK, D), jnp.float32),
            pltpu.VMEM((R, D), jnp.float32),
            pltpu.SemaphoreType.DMA,
            pltpu.SemaphoreType.DMA,
            pltpu.SemaphoreType.DMA,
        ],
    )(_spmm_body)


def kernel(x, edge_index, mask):
    row = edge_index[0].astype(jnp.int32)
    col = edge_index[1].astype(jnp.int32)

    deg = jnp.zeros((N,), jnp.float32).at[col].add(1.0)
    dinv = jnp.where(deg > 0, 1.0 / jnp.sqrt(jnp.maximum(deg, 1e-12)), 0.0)
    w = dinv[row] * dinv[col]

    rs, cs, ws, meta_uu, meta_um = _build_lists(row, col, w, mask)

    x_pad = jnp.zeros((N_PAD, D), jnp.float32).at[:N].set(x)
    zeros_pad = jnp.zeros((N_PAD, D), jnp.float32)

    spmm = _make_spmm()

    b = spmm(zeros_pad, x_pad, cs, rs, ws, meta_um)

    def step(_, h):
        return spmm(b, h, cs, rs, ws, meta_uu)

    h = lax.fori_loop(0, ITERS - 1, step, b)

    return jnp.where(mask[:, None], x, h[:N])


# speedup vs baseline: 12.1164x; 2.1647x over previous
import functools

import jax
import jax.numpy as jnp
from jax import lax
from jax.experimental import pallas as pl
from jax.experimental.pallas import tpu as pltpu
from jax.experimental.pallas import tpu_sc as plsc

N = 10000
E = 160000
D = 256
ITERS = 40

NC = 2
NS = 16
NW = NC * NS
R = 320
N_PAD = NW * R
K = 128
E_CAP = E + NW * K
LANES = 16
SENT = N_PAD
W_SCALE = float(2 ** 30)
EW = 5120
E_S = EW * NW
N_TAB = 10240


def _edge_prep_body(row_hbm, col_hbm, dinv_hbm, maskb_hbm, key_hbm, w30_hbm,
                    row_v, col_v, dinv_v, maskb_v, key_v, w30_v):
    wid = lax.axis_index("s") * NC + lax.axis_index("c")
    base = pl.multiple_of(wid * EW, EW)
    pltpu.sync_copy(dinv_hbm, dinv_v)
    pltpu.sync_copy(maskb_hbm, maskb_v)
    pltpu.sync_copy(row_hbm.at[pl.ds(base, EW)], row_v)
    pltpu.sync_copy(col_hbm.at[pl.ds(base, EW)], col_v)

    lane = jnp.arange(LANES, dtype=jnp.int32)

    @plsc.parallel_loop(0, EW // LANES, unroll=2)
    def group(g):
        o = pl.multiple_of(g * LANES, LANES)
        rv = row_v[pl.ds(o, LANES)]
        cv = col_v[pl.ds(o, LANES)]
        mr = plsc.load_gather(maskb_v, [rv])
        mc = plsc.load_gather(maskb_v, [cv])
        dr = plsc.load_gather(dinv_v, [rv])
        dc = plsc.load_gather(dinv_v, [cv])
        valid = ((base + o) + lane) < E
        key = jnp.where((~valid) | (mr > 0), 2 * N_PAD,
                        jnp.where(mc > 0, N_PAD + rv, rv))
        w30 = (dr * dc * W_SCALE + 0.5).astype(jnp.int32)
        key_v[pl.ds(o, LANES)] = key
        w30_v[pl.ds(o, LANES)] = w30

    pltpu.sync_copy(key_v, key_hbm.at[pl.ds(base, EW)])
    pltpu.sync_copy(w30_v, w30_hbm.at[pl.ds(base, EW)])


def _make_edge_prep():
    mesh = plsc.VectorSubcoreMesh(
        core_axis_name="c", subcore_axis_name="s",
        num_cores=NC, num_subcores=NS)
    return functools.partial(
        pl.kernel,
        out_type=(jax.ShapeDtypeStruct((E_S,), jnp.int32),
                  jax.ShapeDtypeStruct((E_S,), jnp.int32)),
        mesh=mesh,
        scratch_types=[
            pltpu.VMEM((EW,), jnp.int32),
            pltpu.VMEM((EW,), jnp.int32),
            pltpu.VMEM((N_TAB,), jnp.float32),
            pltpu.VMEM((N_TAB,), jnp.int32),
            pltpu.VMEM((EW,), jnp.int32),
            pltpu.VMEM((EW,), jnp.int32),
        ],
        compiler_params=pltpu.CompilerParams(needs_layout_passes=False),
    )(_edge_prep_body)


def _build_lists(key, col_s, w30):
    rs, cs, ws = jax.lax.sort((key, col_s, w30), num_keys=1)
    pad = jnp.full((2 * K,), 2 * N_PAD, jnp.int32)
    rs = jnp.concatenate([rs, pad])
    cs = jnp.concatenate([cs, jnp.zeros((2 * K,), jnp.int32)])
    ws = jnp.concatenate([ws, jnp.zeros((2 * K,), jnp.int32)])

    def meta_for(base):
        bounds = base + jnp.arange(NW + 1, dtype=jnp.int32) * R
        st = jnp.searchsorted(rs, bounds).astype(jnp.int32)
        s, e = st[:NW], st[1:]
        a0 = (s // K) * K
        nch = jnp.where(e > s, (e - a0 + (K - 1)) // K, 0)
        meta = jnp.zeros((NW, LANES), jnp.int32)
        meta = (meta.at[:, 0].set(a0).at[:, 1].set(nch)
                .at[:, 2].set(s).at[:, 3].set(e)
                .at[:, 4].set(bounds[:NW]))
        return meta

    return rs, cs, ws, meta_for(0), meta_for(N_PAD)


HK = K // 2


def _spmm_body(init_hbm, src_hbm, cs_hbm, rs_hbm, ws_hbm, meta_hbm, out_hbm,
               meta_v, ec_v, rows_a, rows_b, acc_v, sem_a, sem_b, sem_e):
    wid = lax.axis_index("s") * NC + lax.axis_index("c")
    r0 = pl.multiple_of(wid * R, R)

    pltpu.sync_copy(meta_hbm, meta_v)
    mrow = meta_v[wid]
    base0 = mrow[0]
    nch = mrow[1]
    seg_s = mrow[2]
    seg_e = mrow[3]
    keyb = mrow[4]

    pltpu.sync_copy(init_hbm.at[pl.ds(r0, R)], acc_v)

    lane = jnp.arange(LANES, dtype=jnp.int32)

    def ec_copies(ci, p):
        off = pl.multiple_of(base0 + ci * K, K)
        return ((cs_hbm.at[pl.ds(off, K)], ec_v.at[p, pl.ds(0, K)]),
                (rs_hbm.at[pl.ds(off, K)], ec_v.at[p, pl.ds(K, K)]),
                (ws_hbm.at[pl.ds(off, K)], ec_v.at[p, pl.ds(2 * K, K)]))

    def ec_start(ci, p):
        for src, dst in ec_copies(ci, p):
            pltpu.async_copy(src, dst, sem_e)

    def ec_wait(ci, p):
        for src, dst in ec_copies(ci, p):
            pltpu.make_async_copy(src, dst, sem_e).wait()

    def gather(p, ho, rows_ref, sem):
        idx = ec_v.at[p, pl.ds(ho, HK)]
        return pltpu.async_copy(src_hbm.at[idx], rows_ref, sem)

    def gather_wait(p, ho, rows_ref, sem):
        idx = ec_v.at[p, pl.ds(ho, HK)]
        pltpu.make_async_copy(src_hbm.at[idx], rows_ref, sem).wait()

    def process(rows_ref, p, ho, base):
        @plsc.parallel_loop(0, HK // LANES, unroll=2)
        def group(g):
            o = pl.multiple_of(g * LANES, LANES)
            jv = (base + ho + o) + lane
            valid = (jv >= seg_s) & (jv < seg_e)
            rsv = ec_v[p, pl.ds(K + ho + o, LANES)]
            rlv = jnp.minimum(jnp.maximum(rsv - keyb, 0), R - 1)
            wv = jnp.where(
                valid,
                ec_v[p, pl.ds(2 * K + ho + o, LANES)].astype(jnp.float32)
                * (1.0 / W_SCALE),
                0.0)
            for j in range(LANES):
                rl = rlv[j]
                wj = wv[j]
                vals = [rows_ref[o + j, pl.ds(dv * LANES, LANES)] * wj
                        for dv in range(D // LANES)]
                for dv in range(D // LANES):
                    sl = pl.ds(dv * LANES, LANES)
                    plsc.addupdate(acc_v.at[rl, sl], vals[dv])

    for src, dst in ec_copies(0, 0):
        pltpu.sync_copy(src, dst)
    gather(0, 0, rows_a, sem_a)
    gather(0, HK, rows_b, sem_b)
    ec_start(1, 1)

    def chunk(ci, carry):
        p = lax.rem(ci, 2)
        pn = 1 - p
        base = pl.multiple_of(base0 + ci * K, K)
        ec_wait(ci + 1, pn)
        gather_wait(p, 0, rows_a, sem_a)
        process(rows_a, p, 0, base)
        gather(pn, 0, rows_a, sem_a)
        gather_wait(p, HK, rows_b, sem_b)
        process(rows_b, p, HK, base)
        gather(pn, HK, rows_b, sem_b)
        ec_start(ci + 2, p)
        return carry

    lax.fori_loop(0, nch, chunk, 0, unroll=False)

    gather_wait(0, 0, rows_a, sem_a)
    gather_wait(0, HK, rows_b, sem_b)
    ec_wait(0, 0)

    pltpu.sync_copy(acc_v, out_hbm.at[pl.ds(r0, R)])


def _make_spmm():
    mesh = plsc.VectorSubcoreMesh(
        core_axis_name="c", subcore_axis_name="s",
        num_cores=NC, num_subcores=NS)
    return functools.partial(
        pl.kernel,
        out_type=jax.ShapeDtypeStruct((N_PAD, D), jnp.float32),
        mesh=mesh,
        scratch_types=[
            pltpu.VMEM((NW, LANES), jnp.int32),
            pltpu.VMEM((2, 3 * K), jnp.int32),
            pltpu.VMEM((HK, D), jnp.float32),
            pltpu.VMEM((HK, D), jnp.float32),
            pltpu.VMEM((R, D), jnp.float32),
            pltpu.SemaphoreType.DMA,
            pltpu.SemaphoreType.DMA,
            pltpu.SemaphoreType.DMA,
        ],
    )(_spmm_body)


def kernel(x, edge_index, mask):
    row = edge_index[0].astype(jnp.int32)
    col = edge_index[1].astype(jnp.int32)

    deg = jnp.zeros((N,), jnp.float32).at[col].add(1.0)
    dinv = jnp.where(deg > 0, 1.0 / jnp.sqrt(jnp.maximum(deg, 1e-12)), 0.0)

    dinv_pad = jnp.zeros((N_TAB,), jnp.float32).at[:N].set(dinv)
    maskb_pad = jnp.zeros((N_TAB,), jnp.int32).at[:N].set(
        mask.astype(jnp.int32))
    row_pad = jnp.zeros((E_S,), jnp.int32).at[:E].set(row)
    col_pad = jnp.zeros((E_S,), jnp.int32).at[:E].set(col)

    key, w30 = _make_edge_prep()(row_pad, col_pad, dinv_pad, maskb_pad)
    rs, cs, ws, meta_uu, meta_um = _build_lists(key, col_pad, w30)

    x_pad = jnp.zeros((N_PAD, D), jnp.float32).at[:N].set(x)
    zeros_pad = jnp.zeros((N_PAD, D), jnp.float32)

    spmm = _make_spmm()

    b = spmm(zeros_pad, x_pad, cs, rs, ws, meta_um)

    def step(_, h):
        return spmm(b, h, cs, rs, ws, meta_uu)

    h = lax.fori_loop(0, ITERS - 1, step, b)

    return jnp.where(mask[:, None], x, h[:N])


# speedup vs baseline: 12.7168x; 1.0495x over previous
import functools

import jax
import jax.numpy as jnp
from jax import lax
from jax.experimental import pallas as pl
from jax.experimental.pallas import tpu as pltpu
from jax.experimental.pallas import tpu_sc as plsc

N = 10000
E = 160000
D = 256
ITERS = 40

NC = 2
NS = 16
NW = NC * NS
R = 320
N_PAD = NW * R
K = 128
E_CAP = E + NW * K
LANES = 16
SENT = N_PAD
W_SCALE = float(2 ** 30)
EW = 5120
E_S = EW * NW
N_TAB = 10240


def _edge_prep_body(row_hbm, col_hbm, dinv_hbm, maskb_hbm, key_hbm, w30_hbm,
                    row_v, col_v, dinv_v, maskb_v, key_v, w30_v):
    wid = lax.axis_index("s") * NC + lax.axis_index("c")
    base = pl.multiple_of(wid * EW, EW)
    pltpu.sync_copy(dinv_hbm, dinv_v)
    pltpu.sync_copy(maskb_hbm, maskb_v)
    pltpu.sync_copy(row_hbm.at[pl.ds(base, EW)], row_v)
    pltpu.sync_copy(col_hbm.at[pl.ds(base, EW)], col_v)

    lane = jnp.arange(LANES, dtype=jnp.int32)

    @plsc.parallel_loop(0, EW // LANES, unroll=2)
    def group(g):
        o = pl.multiple_of(g * LANES, LANES)
        rv = row_v[pl.ds(o, LANES)]
        cv = col_v[pl.ds(o, LANES)]
        mr = plsc.load_gather(maskb_v, [rv])
        mc = plsc.load_gather(maskb_v, [cv])
        dr = plsc.load_gather(dinv_v, [rv])
        dc = plsc.load_gather(dinv_v, [cv])
        valid = ((base + o) + lane) < E
        key = jnp.where((~valid) | (mr > 0), 2 * N_PAD,
                        jnp.where(mc > 0, N_PAD + rv, rv))
        w30 = (dr * dc * W_SCALE + 0.5).astype(jnp.int32)
        key_v[pl.ds(o, LANES)] = key
        w30_v[pl.ds(o, LANES)] = w30

    pltpu.sync_copy(key_v, key_hbm.at[pl.ds(base, EW)])
    pltpu.sync_copy(w30_v, w30_hbm.at[pl.ds(base, EW)])


def _make_edge_prep():
    mesh = plsc.VectorSubcoreMesh(
        core_axis_name="c", subcore_axis_name="s",
        num_cores=NC, num_subcores=NS)
    return functools.partial(
        pl.kernel,
        out_type=(jax.ShapeDtypeStruct((E_S,), jnp.int32),
                  jax.ShapeDtypeStruct((E_S,), jnp.int32)),
        mesh=mesh,
        scratch_types=[
            pltpu.VMEM((EW,), jnp.int32),
            pltpu.VMEM((EW,), jnp.int32),
            pltpu.VMEM((N_TAB,), jnp.float32),
            pltpu.VMEM((N_TAB,), jnp.int32),
            pltpu.VMEM((EW,), jnp.int32),
            pltpu.VMEM((EW,), jnp.int32),
        ],
        compiler_params=pltpu.CompilerParams(needs_layout_passes=False),
    )(_edge_prep_body)


def _build_lists(key, col_s, w30):
    rs, cs, ws = jax.lax.sort((key, col_s, w30), num_keys=1)
    pad = jnp.full((2 * K,), 2 * N_PAD, jnp.int32)
    rs = jnp.concatenate([rs, pad])
    cs = jnp.concatenate([cs, jnp.zeros((2 * K,), jnp.int32)])
    ws = jnp.concatenate([ws, jnp.zeros((2 * K,), jnp.int32)])

    def meta_for(base):
        bounds = base + jnp.arange(NW + 1, dtype=jnp.int32) * R
        st = jnp.searchsorted(rs, bounds).astype(jnp.int32)
        s, e = st[:NW], st[1:]
        a0 = (s // K) * K
        nch = jnp.where(e > s, (e - a0 + (K - 1)) // K, 0)
        meta = jnp.zeros((NW, LANES), jnp.int32)
        meta = (meta.at[:, 0].set(a0).at[:, 1].set(nch)
                .at[:, 2].set(s).at[:, 3].set(e)
                .at[:, 4].set(bounds[:NW]))
        return meta

    return rs, cs, ws, meta_for(0), meta_for(N_PAD)


HK = K // 2


def _spmm_body(init_hbm, src_hbm, cs_hbm, rs_hbm, ws_hbm, meta_hbm, out_hbm,
               meta_v, ec_v, rows_a, rows_b, acc_v, sem_a, sem_b, sem_e):
    wid = lax.axis_index("s") * NC + lax.axis_index("c")
    r0 = pl.multiple_of(wid * R, R)

    pltpu.sync_copy(meta_hbm, meta_v)
    mrow = meta_v[wid]
    base0 = mrow[0]
    nch = mrow[1]
    seg_s = mrow[2]
    seg_e = mrow[3]
    keyb = mrow[4]

    pltpu.sync_copy(init_hbm.at[pl.ds(r0, R)], acc_v)

    lane = jnp.arange(LANES, dtype=jnp.int32)

    def ec_copies(ci, p):
        off = pl.multiple_of(base0 + ci * K, K)
        return ((cs_hbm.at[pl.ds(off, K)], ec_v.at[p, pl.ds(0, K)]),
                (rs_hbm.at[pl.ds(off, K)], ec_v.at[p, pl.ds(K, K)]),
                (ws_hbm.at[pl.ds(off, K)], ec_v.at[p, pl.ds(2 * K, K)]))

    def ec_start(ci, p):
        for src, dst in ec_copies(ci, p):
            pltpu.async_copy(src, dst, sem_e)

    def ec_wait(ci, p):
        for src, dst in ec_copies(ci, p):
            pltpu.make_async_copy(src, dst, sem_e).wait()

    def gather(p, ho, rows_ref, sem):
        idx = ec_v.at[p, pl.ds(ho, HK)]
        return pltpu.async_copy(src_hbm.at[idx], rows_ref, sem)

    def gather_wait(p, ho, rows_ref, sem):
        idx = ec_v.at[p, pl.ds(ho, HK)]
        pltpu.make_async_copy(src_hbm.at[idx], rows_ref, sem).wait()

    def process(rows_ref, p, ho, base):
        @plsc.parallel_loop(0, HK // LANES, unroll=2)
        def group(g):
            o = pl.multiple_of(g * LANES, LANES)
            jv = (base + ho + o) + lane
            valid = (jv >= seg_s) & (jv < seg_e)
            rsv = ec_v[p, pl.ds(K + ho + o, LANES)]
            rlv = jnp.minimum(jnp.maximum(rsv - keyb, 0), R - 1)
            wv = jnp.where(
                valid,
                ec_v[p, pl.ds(2 * K + ho + o, LANES)].astype(jnp.float32)
                * (1.0 / W_SCALE),
                0.0)
            for j in range(LANES):
                rl = rlv[j]
                wj = wv[j]
                vals = [rows_ref[o + j, pl.ds(dv * LANES, LANES)] * wj
                        for dv in range(D // LANES)]
                for dv in range(D // LANES):
                    sl = pl.ds(dv * LANES, LANES)
                    plsc.addupdate(acc_v.at[rl, sl], vals[dv])

    for src, dst in ec_copies(0, 0):
        pltpu.sync_copy(src, dst)
    gather(0, 0, rows_a, sem_a)
    gather(0, HK, rows_b, sem_b)
    ec_start(1, 1)

    def chunk(ci, carry):
        p = lax.rem(ci, 2)
        pn = 1 - p
        base = pl.multiple_of(base0 + ci * K, K)
        ec_wait(ci + 1, pn)
        gather_wait(p, 0, rows_a, sem_a)
        process(rows_a, p, 0, base)
        gather(pn, 0, rows_a, sem_a)
        gather_wait(p, HK, rows_b, sem_b)
        process(rows_b, p, HK, base)
        gather(pn, HK, rows_b, sem_b)
        ec_start(ci + 2, p)
        return carry

    lax.fori_loop(0, nch, chunk, 0, unroll=False)

    gather_wait(0, 0, rows_a, sem_a)
    gather_wait(0, HK, rows_b, sem_b)
    ec_wait(0, 0)

    pltpu.sync_copy(acc_v, out_hbm.at[pl.ds(r0, R)])


def _make_spmm():
    mesh = plsc.VectorSubcoreMesh(
        core_axis_name="c", subcore_axis_name="s",
        num_cores=NC, num_subcores=NS)
    return functools.partial(
        pl.kernel,
        out_type=jax.ShapeDtypeStruct((N_PAD, D), jnp.float32),
        mesh=mesh,
        scratch_types=[
            pltpu.VMEM((NW, LANES), jnp.int32),
            pltpu.VMEM((2, 3 * K), jnp.int32),
            pltpu.VMEM((HK, D), jnp.float32),
            pltpu.VMEM((HK, D), jnp.float32),
            pltpu.VMEM((R, D), jnp.float32),
            pltpu.SemaphoreType.DMA,
            pltpu.SemaphoreType.DMA,
            pltpu.SemaphoreType.DMA,
        ],
    )(_spmm_body)


def kernel(x, edge_index, mask):
    row = edge_index[0].astype(jnp.int32)
    col = edge_index[1].astype(jnp.int32)

    deg = jnp.zeros((N,), jnp.float32).at[col].add(1.0)
    dinv = jnp.where(deg > 0, 1.0 / jnp.sqrt(jnp.maximum(deg, 1e-12)), 0.0)

    dinv_pad = jnp.zeros((N_TAB,), jnp.float32).at[:N].set(dinv)
    maskb_pad = jnp.zeros((N_TAB,), jnp.int32).at[:N].set(
        mask.astype(jnp.int32))
    row_pad = jnp.zeros((E_S,), jnp.int32).at[:E].set(row)
    col_pad = jnp.zeros((E_S,), jnp.int32).at[:E].set(col)

    key, w30 = _make_edge_prep()(row_pad, col_pad, dinv_pad, maskb_pad)
    rs, cs, ws, meta_uu, meta_um = _build_lists(key, col_pad, w30)

    x_pad = jnp.zeros((N_PAD, D), jnp.float32).at[:N].set(x)
    zeros_pad = jnp.zeros((N_PAD, D), jnp.float32)

    spmm = _make_spmm()

    b = spmm(zeros_pad, x_pad, cs, rs, ws, meta_um)

    h = b
    for _ in range(ITERS - 1):
        h = spmm(b, h, cs, rs, ws, meta_uu)

    return jnp.where(mask[:, None], x, h[:N])
